# Initial kernel scaffold; baseline (speedup 1.0000x reference)
import jax
import jax.numpy as jnp
from jax.experimental import pallas as pl


def kernel(x, edge_index, embed, msg_w1, msg_b1, msg_w2, msg_b2, self_w, self_b, out_w1, out_b1, out_w2, out_b2):
    raise NotImplementedError("write your pallas kernel here")


# speedup vs baseline: 1.4786x; 1.4786x over previous
import functools

import jax
import jax.numpy as jnp
from jax import lax
from jax.experimental import pallas as pl
from jax.experimental.pallas import tpu as pltpu
from jax.experimental.pallas import tpu_sc as plsc

_N = 50000
_E = 800000
_D = 64
_V = 120
_EP = 819200
_RPSE = 800
_RPSD = 400
_CH = 64
_NPS = 3128
_NPSL = _N - 15 * _NPS
_NPAD = _N + 8
_BLK = 2000
_F32 = jnp.float32
_BF = jnp.bfloat16

_mesh = plsc.VectorSubcoreMesh(core_axis_name="c", subcore_axis_name="s")


@functools.partial(
    pl.kernel,
    out_type=jax.ShapeDtypeStruct((_EP, _D), _F32),
    mesh=_mesh,
    scratch_types=[
        pltpu.VMEM((_CH,), jnp.int32),
        pltpu.VMEM((_CH,), jnp.int32),
        pltpu.VMEM((_CH, 128), _F32),
        pltpu.VMEM((_CH, 128), _F32),
        pltpu.VMEM((_CH, _D), _F32),
        pltpu.SemaphoreType.DMA,
        pltpu.SemaphoreType.DMA,
    ],
)
def _edge_kernel(tab, src2, dst2, out, idxs, idxd, bufa, bufb, bufr,
                 sema, semb):
    cid = lax.axis_index("c")
    sid = lax.axis_index("s")
    zv = jnp.zeros((16,), _F32)
    wid = cid * 16 + sid
    ebase = wid * (_EP // 32)

    @pl.loop(0, _EP // 32 // _CH)
    def _(bk):
        e0 = ebase + bk * _CH
        pltpu.sync_copy(src2.at[pl.ds(e0, _CH)], idxs)
        pltpu.sync_copy(dst2.at[pl.ds(e0, _CH)], idxd)
        cpa = pltpu.async_copy(tab.at[idxs], bufa, sema)
        cpb = pltpu.async_copy(tab.at[idxd], bufb, semb)
        cpa.wait()
        cpb.wait()

        @pl.loop(0, _CH)
        def _(r):
            for q in range(4):
                a = bufa[r, pl.ds(16 * q, 16)]
                b = bufb[r, pl.ds(_D + 16 * q, 16)]
                bufr[r, pl.ds(16 * q, 16)] = jnp.maximum(a + b, zv)

        pltpu.sync_copy(bufr, out.at[pl.ds(e0, _CH)])


def _embed_body(x_ref, emb_ref, wc_ref, bc_ref, h_ref, tab_ref):
    oh = (x_ref[...] == lax.broadcasted_iota(jnp.int32, (_BLK, _V), 1))
    h = jnp.dot(oh.astype(_F32), emb_ref[...], preferred_element_type=_F32)
    h_ref[...] = h
    tab_ref[...] = jnp.dot(h, wc_ref[...],
                           preferred_element_type=_F32) + bc_ref[...]


def _embed_call(x2, emb, wc, bc):
    return pl.pallas_call(
        _embed_body,
        grid=(_N // _BLK,),
        in_specs=[
            pl.BlockSpec((_BLK, 1), lambda i: (i, 0)),
            pl.BlockSpec((_V, _D), lambda i: (0, 0)),
            pl.BlockSpec((_D, 2 * _D), lambda i: (0, 0)),
            pl.BlockSpec((1, 2 * _D), lambda i: (0, 0)),
        ],
        out_specs=[
            pl.BlockSpec((_BLK, _D), lambda i: (i, 0)),
            pl.BlockSpec((_BLK, 2 * _D), lambda i: (i, 0)),
        ],
        out_shape=[
            jax.ShapeDtypeStruct((_N, _D), _F32),
            jax.ShapeDtypeStruct((_NPAD, 2 * _D), _F32),
        ],
    )(x2, emb, wc, bc)


def _node_update(h_ref, p_ref, dg_ref, sw_ref, sb_ref, w2_ref, b2_ref):
    agg = jnp.dot(p_ref[...], w2_ref[...], preferred_element_type=_F32) \
        + dg_ref[...] * b2_ref[...]
    return jnp.maximum(
        jnp.dot(h_ref[...], sw_ref[...], preferred_element_type=_F32)
        + sb_ref[...] + agg, 0.0)


def _update_body(h_ref, p_ref, dg_ref, sw_ref, sb_ref, w2_ref, b2_ref,
                 wc_ref, bc_ref, ho_ref, tab_ref):
    hn = _node_update(h_ref, p_ref, dg_ref, sw_ref, sb_ref, w2_ref, b2_ref)
    ho_ref[...] = hn
    tab_ref[...] = jnp.dot(hn, wc_ref[...],
                           preferred_element_type=_F32) + bc_ref[...]


def _update_call(h, p2, dg2, sw, sb, w2, b2, wc, bc):
    return pl.pallas_call(
        _update_body,
        grid=(_N // _BLK,),
        in_specs=[
            pl.BlockSpec((_BLK, _D), lambda i: (i, 0)),
            pl.BlockSpec((_BLK, _D), lambda i: (i, 0)),
            pl.BlockSpec((_BLK, 1), lambda i: (i, 0)),
            pl.BlockSpec((_D, _D), lambda i: (0, 0)),
            pl.BlockSpec((1, _D), lambda i: (0, 0)),
            pl.BlockSpec((_D, _D), lambda i: (0, 0)),
            pl.BlockSpec((1, _D), lambda i: (0, 0)),
            pl.BlockSpec((_D, 2 * _D), lambda i: (0, 0)),
            pl.BlockSpec((1, 2 * _D), lambda i: (0, 0)),
        ],
        out_specs=[
            pl.BlockSpec((_BLK, _D), lambda i: (i, 0)),
            pl.BlockSpec((_BLK, 2 * _D), lambda i: (i, 0)),
        ],
        out_shape=[
            jax.ShapeDtypeStruct((_N, _D), _F32),
            jax.ShapeDtypeStruct((_NPAD, 2 * _D), _F32),
        ],
    )(h, p2, dg2, sw, sb, w2, b2, wc, bc)


def _head_body(h_ref, p_ref, dg_ref, sw_ref, sb_ref, w2_ref, b2_ref,
               ow1_ref, ob1_ref, ow2_ref, ob2_ref, o_ref, acc_ref):
    i = pl.program_id(0)
    hn = _node_update(h_ref, p_ref, dg_ref, sw_ref, sb_ref, w2_ref, b2_ref)

    @pl.when(i == 0)
    def _():
        acc_ref[...] = jnp.zeros_like(acc_ref)

    acc_ref[...] += jnp.sum(hn, axis=0, keepdims=True)

    @pl.when(i == pl.num_programs(0) - 1)
    def _():
        g = acc_ref[...] * (1.0 / _N)
        t = jnp.maximum(
            jnp.dot(g, ow1_ref[...], preferred_element_type=_F32)
            + ob1_ref[...], 0.0)
        o_ref[...] = jnp.dot(t, ow2_ref[...], preferred_element_type=_F32) \
            + ob2_ref[...]


def _head_call(h, p2, dg2, sw, sb, w2, b2, ow1, ob1, ow2, ob2):
    return pl.pallas_call(
        _head_body,
        grid=(_N // _BLK,),
        in_specs=[
            pl.BlockSpec((_BLK, _D), lambda i: (i, 0)),
            pl.BlockSpec((_BLK, _D), lambda i: (i, 0)),
            pl.BlockSpec((_BLK, 1), lambda i: (i, 0)),
            pl.BlockSpec((_D, _D), lambda i: (0, 0)),
            pl.BlockSpec((1, _D), lambda i: (0, 0)),
            pl.BlockSpec((_D, _D), lambda i: (0, 0)),
            pl.BlockSpec((1, _D), lambda i: (0, 0)),
            pl.BlockSpec((_D, _D), lambda i: (0, 0)),
            pl.BlockSpec((1, _D), lambda i: (0, 0)),
            pl.BlockSpec((_D, 1), lambda i: (0, 0)),
            pl.BlockSpec((1, 1), lambda i: (0, 0)),
        ],
        out_specs=pl.BlockSpec((1, 1), lambda i: (0, 0)),
        out_shape=jax.ShapeDtypeStruct((1, 1), _F32),
        scratch_shapes=[pltpu.VMEM((1, _D), _F32)],
    )(h, p2, dg2, sw, sb, w2, b2, ow1, ob1, ow2, ob2)


def _layer_tables(msg_w1_l, msg_b1_l):
    wc = jnp.concatenate([msg_w1_l[:_D], msg_w1_l[_D:]], axis=1)
    bc = jnp.concatenate([jnp.zeros((_D,), _F32),
                          msg_b1_l]).reshape(1, 2 * _D)
    return wc, bc


def kernel(x, edge_index, embed, msg_w1, msg_b1, msg_w2, msg_b2,
           self_w, self_b, out_w1, out_b1, out_w2, out_b2):
    nl = msg_w1.shape[0]
    x2 = x.reshape(_N, 1).astype(jnp.int32)
    pad = _EP - _E
    src_i = edge_index[0].astype(jnp.int32)
    dst_i = edge_index[1].astype(jnp.int32)
    src2 = jnp.concatenate([src_i, jnp.zeros((pad,), jnp.int32)])
    dst2 = jnp.concatenate([dst_i, jnp.full((pad,), _N, jnp.int32)])

    deg = jax.ops.segment_sum(jnp.ones((_E,), _F32), dst_i,
                              num_segments=_N).reshape(_N, 1)
    wc, bc = _layer_tables(msg_w1[0], msg_b1[0])
    h, tab = _embed_call(x2, embed, wc, bc)
    for l in range(nl):
        r = _edge_kernel(tab, src2, dst2)
        p2 = jax.ops.segment_sum(r[:_E], dst_i, num_segments=_N)
        args = (h, p2, deg, self_w[l], self_b[l].reshape(1, _D),
                msg_w2[l], msg_b2[l].reshape(1, _D))
        if l + 1 < nl:
            wc, bc = _layer_tables(msg_w1[l + 1], msg_b1[l + 1])
            h, tab = _update_call(*args, wc, bc)
        else:
            return _head_call(*args, out_w1, out_b1.reshape(1, _D),
                              out_w2, out_b2.reshape(1, 1))
